# baseline (speedup 1.0000x reference)
import functools

import jax
import jax.numpy as jnp
from jax import lax
from jax.experimental import pallas as pl
from jax.experimental.pallas import tpu as pltpu
from jax.experimental.pallas import tpu_sc as plsc

B = 8
H = W = 112
C = 96
F = H * W * C
UPS = 2
OH = H * UPS
OW = W * UPS
PLANE = OW * C
S = OH * PLANE

NC = 2
NS = 16
L = 16

NCHUNK = 6
P38 = 38
P36 = 36
CHUNK38 = P38 * PLANE
CHUNK36 = P36 * PLANE
DUMP = 2048
ACCH = CHUNK38 + DUMP
NTASK = B * NCHUNK

PER_TILE = F // NS
PIECE = 2688
NPIECE = PER_TILE // PIECE
PC = PIECE + 128
HOOK = 8
WB38 = 399 * 128
WB36 = 378 * 128
ZW = 2688


def _body(feat_hbm, idx_hbm, out_hbm, idx_v, feat_v, adjc, featc, zero_v,
          acc_sh, sem_in, sem_sc, sem_z, sem_wb):
    core = lax.axis_index("c")
    tile = lax.axis_index("s")
    iota = lax.iota(jnp.int32, L)

    def _zinit(g, _):
        zero_v[pl.ds(g * L, L)] = jnp.zeros((L,), jnp.float32)
        return 0

    lax.fori_loop(0, ZW // L, _zinit, 0)

    def _stage_start(p, par, in_base):
        src = in_base + p * PIECE
        pltpu.async_copy(idx_hbm.at[pl.ds(src, PIECE)],
                         idx_v.at[pl.ds(par * PIECE, PIECE)], sem_in)
        pltpu.async_copy(feat_hbm.at[pl.ds(src, PIECE)],
                         feat_v.at[pl.ds(par * PIECE, PIECE)], sem_in)

    def _stage_wait(p, par, in_base):
        src = in_base + p * PIECE
        pltpu.make_async_copy(idx_hbm.at[pl.ds(src, PIECE)],
                              idx_v.at[pl.ds(par * PIECE, PIECE)],
                              sem_in).wait()
        pltpu.make_async_copy(feat_hbm.at[pl.ds(src, PIECE)],
                              feat_v.at[pl.ds(par * PIECE, PIECE)],
                              sem_in).wait()

    def _compact(par, shift, lo, hi):
        def _row(j, wptr):
            locs, oks, vs, pcs = [], [], [], []
            for g in range(128 // L):
                off = par * PIECE + j * 128 + g * L
                raw = idx_v[pl.ds(off, L)]
                ls = raw - shift
                ok = (ls >= lo) & (ls < hi)
                locs.append(ls)
                oks.append(ok)
                vs.append(feat_v[pl.ds(off, L)])
                pcs.append(plsc.all_reduce_population_count(ok)[0])
            offs = [wptr]
            for g in range(128 // L):
                offs.append(offs[g] + pcs[g])
            for g in range(128 // L):
                plsc.store_compressed(adjc.at[pl.ds(par * PC + offs[g], L)],
                                      locs[g], mask=oks[g])
                plsc.store_compressed(
                    featc.at[pl.ds(par * PIECE + offs[g], L)],
                    vs[g], mask=oks[g])
            return offs[128 // L]

        return lax.fori_loop(0, PIECE // 128, _row, jnp.int32(0))

    def _pad(par, cnt, p, dump_base):
        for q in range(8):
            offs = (p * 256 + q * 16 + tile * 64) & (DUMP - 16)
            adjc[pl.ds(par * PC + cnt + q * L, L)] = dump_base + offs + iota

    def _scatter_fire(par, rows):
        def _row(j, _):
            pltpu.async_copy(
                featc.at[pl.ds(par * PIECE + j * 128, 128)],
                acc_sh.at[adjc.at[pl.ds(par * PC + j * 128, 128)]],
                sem_sc, add=True)
            return 0

        lax.fori_loop(0, rows, _row, 0)

    def _scatter_drain(par, rows):
        def _row(j, _):
            pltpu.make_async_copy(
                featc.at[pl.ds(par * PIECE + j * 128, 128)],
                acc_sh.at[adjc.at[pl.ds(par * PC + j * 128, 128)]],
                sem_sc).wait()
            return 0

        lax.fori_loop(0, rows, _row, 0)

    def _task_params(idx):
        t = idx * NC + core
        b = t // NCHUNK
        k = t % NCHUNK
        hoff = lax.rem(idx, 2) * ACCH
        base_k = jnp.where(k < 4, k * P38, 4 * P38 + (k - 4) * P36) * PLANE
        return b, k, hoff, base_k

    def _wb_fire(b, k, hoff, base_k):
        @pl.when(k < 4)
        def _():
            off = tile * WB38
            pltpu.async_copy(acc_sh.at[pl.ds(hoff + off, WB38)],
                             out_hbm.at[pl.ds(b * S + base_k + off, WB38)],
                             sem_wb)

        @pl.when(k >= 4)
        def _():
            off = tile * WB36
            pltpu.async_copy(acc_sh.at[pl.ds(hoff + off, WB36)],
                             out_hbm.at[pl.ds(b * S + base_k + off, WB36)],
                             sem_wb)

    def _wb_drain(b, k, hoff, base_k):
        @pl.when(k < 4)
        def _():
            off = tile * WB38
            pltpu.make_async_copy(
                acc_sh.at[pl.ds(hoff + off, WB38)],
                out_hbm.at[pl.ds(b * S + base_k + off, WB38)],
                sem_wb).wait()

        @pl.when(k >= 4)
        def _():
            off = tile * WB36
            pltpu.make_async_copy(
                acc_sh.at[pl.ds(hoff + off, WB36)],
                out_hbm.at[pl.ds(b * S + base_k + off, WB36)],
                sem_wb).wait()

    def _zero_fire(k, hoff):
        @pl.when(k < 4)
        def _():
            def _zf(q, _):
                pltpu.async_copy(
                    zero_v, acc_sh.at[pl.ds(hoff + tile * WB38 + q * ZW, ZW)],
                    sem_z)
                return 0

            lax.fori_loop(0, WB38 // ZW, _zf, 0)

        @pl.when(k >= 4)
        def _():
            def _zf(q, _):
                pltpu.async_copy(
                    zero_v, acc_sh.at[pl.ds(hoff + tile * WB36 + q * ZW, ZW)],
                    sem_z)
                return 0

            lax.fori_loop(0, WB36 // ZW, _zf, 0)
            pltpu.async_copy(
                zero_v, acc_sh.at[pl.ds(hoff + CHUNK36 + tile * ZW, ZW)],
                sem_z)

    def _zero_drain(k, hoff):
        @pl.when(k < 4)
        def _():
            def _zd(q, _):
                pltpu.make_async_copy(
                    zero_v, acc_sh.at[pl.ds(hoff + tile * WB38 + q * ZW, ZW)],
                    sem_z).wait()
                return 0

            lax.fori_loop(0, WB38 // ZW, _zd, 0)

        @pl.when(k >= 4)
        def _():
            def _zd(q, _):
                pltpu.make_async_copy(
                    zero_v, acc_sh.at[pl.ds(hoff + tile * WB36 + q * ZW, ZW)],
                    sem_z).wait()
                return 0

            lax.fori_loop(0, WB36 // ZW, _zd, 0)
            pltpu.make_async_copy(
                zero_v, acc_sh.at[pl.ds(hoff + CHUNK36 + tile * ZW, ZW)],
                sem_z).wait()

    for hh in range(2):
        def _zf(q, _, hh=hh):
            pltpu.async_copy(
                zero_v,
                acc_sh.at[pl.ds(hh * ACCH + tile * WB38 + q * ZW, ZW)],
                sem_z)
            return 0

        lax.fori_loop(0, WB38 // ZW, _zf, 0)
    for hh in range(2):
        def _zd(q, _, hh=hh):
            pltpu.make_async_copy(
                zero_v,
                acc_sh.at[pl.ds(hh * ACCH + tile * WB38 + q * ZW, ZW)],
                sem_z).wait()
            return 0

        lax.fori_loop(0, WB38 // ZW, _zd, 0)
    plsc.subcore_barrier()

    def _task(i, _):
        b, k, hoff, base_k = _task_params(i)
        bp, kp, hpoff, base_kp = _task_params(i - 1)
        chunk_size = jnp.where(k < 4, CHUNK38, CHUNK36)
        shift = base_k - hoff
        lo = hoff
        hi = hoff + chunk_size
        dump_base = hoff + CHUNK38
        in_base = b * F + tile * PER_TILE

        _stage_start(0, 0, in_base)

        def _piece(p, rows_prev):
            cur = lax.rem(p, 2)
            nxt = 1 - cur
            _stage_wait(p, cur, in_base)
            cnt = _compact(cur, shift, lo, hi)
            _pad(cur, cnt, p, dump_base)
            rows = (cnt + 127) // 128

            @pl.when(p > 0)
            def _():
                _scatter_drain(nxt, rows_prev)

            _scatter_fire(cur, rows)

            @pl.when(p + 1 < NPIECE)
            def _():
                _stage_start(p + 1, nxt, in_base)

            @pl.when((p == HOOK) & (i >= 1))
            def _():
                _wb_drain(bp, kp, hpoff, base_kp)
                _zero_fire(kp, hpoff)

            return rows

        rows_last = lax.fori_loop(0, NPIECE, _piece, jnp.int32(0))
        _scatter_drain((NPIECE - 1) % 2, rows_last)

        @pl.when(i >= 1)
        def _():
            _zero_drain(kp, hpoff)

        plsc.subcore_barrier()
        _wb_fire(b, k, hoff, base_k)
        return 0

    lax.fori_loop(0, NTASK // NC, _task, 0)

    bl, kl, hloff, base_kl = _task_params(NTASK // NC - 1)
    _wb_drain(bl, kl, hloff, base_kl)


@functools.partial(
    pl.kernel,
    out_type=jax.ShapeDtypeStruct((B * S,), jnp.float32),
    mesh=plsc.VectorSubcoreMesh(core_axis_name="c", subcore_axis_name="s"),
    scratch_types=[
        pltpu.VMEM((2 * PIECE,), jnp.int32),
        pltpu.VMEM((2 * PIECE,), jnp.float32),
        pltpu.VMEM((2 * PC,), jnp.int32),
        pltpu.VMEM((2 * PIECE,), jnp.float32),
        pltpu.VMEM((ZW,), jnp.float32),
        pltpu.VMEM_SHARED((2 * ACCH,), jnp.float32),
        pltpu.SemaphoreType.DMA,
        pltpu.SemaphoreType.DMA,
        pltpu.SemaphoreType.DMA,
        pltpu.SemaphoreType.DMA,
    ],
    compiler_params=pltpu.CompilerParams(needs_layout_passes=False),
)
def _scatter_add_kernel(feat_hbm, idx_hbm, out_hbm, idx_v, feat_v, adjc,
                        featc, zero_v, acc_sh, sem_in, sem_sc, sem_z, sem_wb):
    _body(feat_hbm, idx_hbm, out_hbm, idx_v, feat_v, adjc, featc, zero_v,
          acc_sh, sem_in, sem_sc, sem_z, sem_wb)


def kernel(features, indices):
    feat_flat = features.reshape(B * F)
    idx = indices.reshape(B * F).astype(jnp.int32)
    oh = idx // PLANE
    r = idx - oh * PLANE
    ow = r // C
    oc = r - ow * C
    ridx = oh * PLANE + oc * OW + ow
    out = _scatter_add_kernel(feat_flat, ridx)
    return out.reshape(B, OH, C, OW).transpose(0, 1, 3, 2)
